# Initial kernel scaffold; baseline (speedup 1.0000x reference)
import jax
import jax.numpy as jnp
from jax.experimental import pallas as pl


def kernel(hidden_states, gate_w, ws, w2s):
    raise NotImplementedError("write your pallas kernel here")


# speedup vs baseline: 1.4359x; 1.4359x over previous
import functools

import jax
import jax.numpy as jnp
from jax.experimental import pallas as pl
from jax.experimental.pallas import tpu as pltpu

T, H, I, E, TOP_K = 4096, 1024, 2048, 8, 2

BT = 1024
BN = 1024
N_CHUNKS = I // BN


def _dense_body(x_ref, gw_ref, wsg_ref, wsu_ref, w2_ref, out_ref):
    e = pl.program_id(1)
    n = pl.program_id(2)

    @pl.when(jnp.logical_and(e == 0, n == 0))
    def _():
        out_ref[...] = jnp.zeros_like(out_ref)

    xb = x_ref[...].astype(jnp.bfloat16)

    logits = jax.lax.dot_general(
        xb, gw_ref[...].astype(jnp.bfloat16),
        (((1,), (1,)), ((), ())), preferred_element_type=jnp.float32)
    lmax = jnp.max(logits, axis=1, keepdims=True)
    p = jnp.exp(logits - lmax)
    lane = jax.lax.broadcasted_iota(jnp.int32, (BT, E), 1)
    m1 = jnp.max(p, axis=1, keepdims=True)
    i1 = jnp.min(jnp.where(p == m1, lane, E), axis=1, keepdims=True)
    p2 = jnp.where(lane == i1, -jnp.inf, p)
    m2 = jnp.max(p2, axis=1, keepdims=True)
    i2 = jnp.min(jnp.where(p2 == m2, lane, E), axis=1, keepdims=True)
    denom = m1 + m2
    wcol = jnp.where(i1 == e, m1, jnp.where(i2 == e, m2, 0.0)) / denom

    g = jax.lax.dot_general(
        xb, wsg_ref[0].astype(jnp.bfloat16),
        (((1,), (1,)), ((), ())), preferred_element_type=jnp.float32)
    u = jax.lax.dot_general(
        xb, wsu_ref[0].astype(jnp.bfloat16),
        (((1,), (1,)), ((), ())), preferred_element_type=jnp.float32)
    h = (g * (1.0 / (1.0 + jnp.exp(-g))) * u).astype(jnp.bfloat16)
    part = jax.lax.dot_general(
        h, w2_ref[0].astype(jnp.bfloat16),
        (((1,), (1,)), ((), ())), preferred_element_type=jnp.float32)
    out_ref[...] += wcol * part


@jax.jit
def kernel(hidden_states, gate_w, ws, w2s):
    grid = (T // BT, E, N_CHUNKS)
    return pl.pallas_call(
        _dense_body,
        grid=grid,
        in_specs=[
            pl.BlockSpec((BT, H), lambda t, e, n: (t, 0)),
            pl.BlockSpec((E, H), lambda t, e, n: (0, 0)),
            pl.BlockSpec((1, BN, H), lambda t, e, n: (e, n, 0)),
            pl.BlockSpec((1, BN, H), lambda t, e, n: (e, N_CHUNKS + n, 0)),
            pl.BlockSpec((1, H, BN), lambda t, e, n: (e, 0, n)),
        ],
        out_specs=pl.BlockSpec((BT, H), lambda t, e, n: (t, 0)),
        out_shape=jax.ShapeDtypeStruct((T, H), jnp.float32),
    )(hidden_states, gate_w, ws, ws, w2s)
